# Initial kernel scaffold; baseline (speedup 1.0000x reference)
import functools

import jax
import jax.numpy as jnp
from jax import lax
from jax.experimental import pallas as pl
from jax.experimental.pallas import tpu as pltpu
from jax.experimental.pallas import tpu_sc as plsc

N = 10000
E = 320000
T = 12
D = 128
H = 128

NPAD = 10240
NC = 2
NS = 16
HD = D // NC

CH = 80
EPT = E // NS
NCHUNK = EPT // CH
EPW = E // (NC * NS)
NCHUNK_D = EPW // CH

ROWS_PT = NPAD // NS
ZROWS = 160

_SC_MESH = plsc.VectorSubcoreMesh(core_axis_name="c", subcore_axis_name="s")


def _deg_body(src_hbm, dst_hbm, out_hbm, my_src, my_dst, ones_v, zb,
              acc_s, acc_d, sem):
    c = lax.axis_index("c")
    s = lax.axis_index("s")

    pltpu.sync_copy(src_hbm.at[c, s], my_src)
    pltpu.sync_copy(dst_hbm.at[c, s], my_dst)

    one16 = jnp.ones((16,), jnp.float32)
    zero16 = jnp.zeros((16,), jnp.float32)
    for k in range(CH // 16):
        ones_v[pl.ds(k * 16, 16)] = one16

    def _zb(i, _):
        zb[pl.ds(i * 16, 16)] = zero16
        return 0
    lax.fori_loop(0, ROWS_PT // 16, _zb, 0)

    pltpu.sync_copy(zb, acc_s.at[pl.ds(s * ROWS_PT, ROWS_PT)])
    pltpu.sync_copy(zb, acc_d.at[pl.ds(s * ROWS_PT, ROWS_PT)])
    plsc.subcore_barrier()

    def _chunk(g, _):
        ds_ = []
        for u in range(5):
            j = g * 5 + u
            ds_.append(pltpu.async_copy(ones_v, acc_s.at[my_src.at[j]], sem,
                                        add=True))
            ds_.append(pltpu.async_copy(ones_v, acc_d.at[my_dst.at[j]], sem,
                                        add=True))
        for d in ds_:
            d.wait()
        return 0
    lax.fori_loop(0, NCHUNK_D // 5, _chunk, 0)

    plsc.subcore_barrier()
    pltpu.sync_copy(acc_s.at[pl.ds(s * ROWS_PT, ROWS_PT)],
                    out_hbm.at[c, 0, pl.ds(s * ROWS_PT, ROWS_PT)])
    pltpu.sync_copy(acc_d.at[pl.ds(s * ROWS_PT, ROWS_PT)],
                    out_hbm.at[c, 1, pl.ds(s * ROWS_PT, ROWS_PT)])


def _sc_degrees(src_r, dst_r):
    return pl.kernel(
        _deg_body,
        out_type=jax.ShapeDtypeStruct((NC, 2, NPAD), jnp.float32),
        mesh=_SC_MESH,
        scratch_types=[
            pltpu.VMEM((NCHUNK_D, CH), jnp.int32),
            pltpu.VMEM((NCHUNK_D, CH), jnp.int32),
            pltpu.VMEM((CH,), jnp.float32),
            pltpu.VMEM((ROWS_PT,), jnp.float32),
            pltpu.VMEM_SHARED((NPAD,), jnp.float32),
            pltpu.VMEM_SHARED((NPAD,), jnp.float32),
            pltpu.SemaphoreType.DMA,
        ],
    )(src_r, dst_r)


def _segsum_body(xflat, src_hbm, dst_hbm, out_hbm, my_src, my_dst,
                 stage0, stage1, rows0, rows1, zb, acc, sem0, sem1):
    c = lax.axis_index("c")
    s = lax.axis_index("s")

    pltpu.sync_copy(src_hbm.at[s], my_src)
    pltpu.sync_copy(dst_hbm.at[s], my_dst)

    zero16 = jnp.zeros((16,), jnp.float32)

    def _zb(i, _):
        r = i // (HD // 16)
        k = i % (HD // 16)
        zb[r, pl.ds(k * 16, 16)] = zero16
        return 0
    lax.fori_loop(0, ZROWS * (HD // 16), _zb, 0)

    def _stage_fire(j, base, stage, rows, sem):
        for k in range(CH // 16):
            v = my_src[j, pl.ds(k * 16, 16)]
            stage[pl.ds(k * 16, 16)] = v + base
        return pltpu.async_copy(xflat.at[stage], rows, sem)

    def _per_t(t, _):
        base = (c * T + t) * NPAD
        for k in range(ROWS_PT // ZROWS):
            pltpu.sync_copy(zb, acc.at[pl.ds(s * ROWS_PT + k * ZROWS, ZROWS), :])
        plsc.subcore_barrier()

        _stage_fire(0, base, stage0, rows0, sem0)

        def _pair(m, _):
            j0 = 2 * m
            _stage_fire(j0 + 1, base, stage1, rows1, sem1)
            pltpu.make_async_copy(xflat.at[stage0], rows0, sem0).wait()
            pltpu.sync_copy(rows0, acc.at[my_dst.at[j0]], add=True)

            @pl.when(m < NCHUNK // 2 - 1)
            def _():
                _stage_fire(j0 + 2, base, stage0, rows0, sem0)

            pltpu.make_async_copy(xflat.at[stage1], rows1, sem1).wait()
            pltpu.sync_copy(rows1, acc.at[my_dst.at[j0 + 1]], add=True)
            return 0

        lax.fori_loop(0, NCHUNK // 2, _pair, 0)
        plsc.subcore_barrier()
        pltpu.sync_copy(
            acc.at[pl.ds(s * ROWS_PT, ROWS_PT), :],
            out_hbm.at[t, pl.ds(s * ROWS_PT, ROWS_PT), pl.ds(c * HD, HD)])
        return 0

    lax.fori_loop(0, T, _per_t, 0)


def _sc_segsum(xflat, src_r, dst_r):
    return pl.kernel(
        _segsum_body,
        out_type=jax.ShapeDtypeStruct((T, NPAD, D), jnp.float32),
        mesh=_SC_MESH,
        scratch_types=[
            pltpu.VMEM((NCHUNK, CH), jnp.int32),
            pltpu.VMEM((NCHUNK, CH), jnp.int32),
            pltpu.VMEM((CH,), jnp.int32),
            pltpu.VMEM((CH,), jnp.int32),
            pltpu.VMEM((CH, HD), jnp.float32),
            pltpu.VMEM((CH, HD), jnp.float32),
            pltpu.VMEM((ZROWS, HD), jnp.float32),
            pltpu.VMEM_SHARED((NPAD, HD), jnp.float32),
            pltpu.SemaphoreType.DMA,
            pltpu.SemaphoreType.DMA,
        ],
    )(xflat, src_r, dst_r)


def _norms_body(deg_ref, out_ref):
    d = deg_ref[0] + deg_ref[1]
    out_ref[...] = lax.rsqrt(jnp.maximum(d, 1.0))


def _tc_norms(deg):
    return pl.pallas_call(
        _norms_body,
        out_shape=jax.ShapeDtypeStruct((2, NPAD), jnp.float32),
    )(deg)


BN = 2000
BP = 2048


def _prep_body(h_ref, ns_ref, out_ref):
    out_ref[0, 0] = h_ref[:, 0, :] * ns_ref[...]


def _tc_prep(h, norm_src_col):
    grid = (NC, T, N // BN)
    return pl.pallas_call(
        _prep_body,
        grid=grid,
        in_specs=[
            pl.BlockSpec((BN, 1, HD), lambda c, t, n: (n, t, c)),
            pl.BlockSpec((BN, 1), lambda c, t, n: (n, 0)),
        ],
        out_specs=pl.BlockSpec((1, 1, BN, HD), lambda c, t, n: (c, t, n, 0)),
        out_shape=jax.ShapeDtypeStruct((NC, T, NPAD, HD), jnp.float32),
    )(h, norm_src_col)


def _gelu(x):
    return 0.5 * x * (1.0 + lax.erf(x * 0.7071067811865476))


def _layer1_body(agg_ref, nd_ref, ns_ref, w_ref, b_ref, out_ref):
    z = agg_ref[0] * nd_ref[...]
    y = jnp.dot(z, w_ref[...], preferred_element_type=jnp.float32)
    y = _gelu(y + b_ref[...])
    out_ref[0, 0] = y * ns_ref[...]


def _tc_layer1(agg1, nd_col, ns_col, W1, b1):
    grid = (NC, T, NPAD // BP)
    return pl.pallas_call(
        _layer1_body,
        grid=grid,
        in_specs=[
            pl.BlockSpec((1, BP, D), lambda c, t, n: (t, n, 0)),
            pl.BlockSpec((BP, 1), lambda c, t, n: (n, 0)),
            pl.BlockSpec((BP, 1), lambda c, t, n: (n, 0)),
            pl.BlockSpec((D, HD), lambda c, t, n: (0, c)),
            pl.BlockSpec((1, HD), lambda c, t, n: (0, c)),
        ],
        out_specs=pl.BlockSpec((1, 1, BP, HD), lambda c, t, n: (c, t, n, 0)),
        out_shape=jax.ShapeDtypeStruct((NC, T, NPAD, HD), jnp.float32),
    )(agg1, nd_col, ns_col, W1, b1)


def _layer2_body(agg_ref, nd_ref, w_ref, b_ref, out_ref):
    nb = pl.program_id(1)
    z = agg_ref[0] * nd_ref[...]
    y = jnp.dot(z, w_ref[...], preferred_element_type=jnp.float32)
    y = _gelu(y + b_ref[...])
    row = nb * BP + lax.broadcasted_iota(jnp.int32, (BP, 1), 0)
    y = jnp.where(row < N, y, 0.0)
    part = jnp.sum(y, axis=0, keepdims=True)

    @pl.when(nb == 0)
    def _():
        out_ref[...] = jnp.zeros_like(out_ref)

    out_ref[...] += part * (1.0 / N)


def _tc_layer2(agg2, nd_col, W2, b2):
    grid = (T, NPAD // BP)
    return pl.pallas_call(
        _layer2_body,
        grid=grid,
        in_specs=[
            pl.BlockSpec((1, BP, D), lambda t, n: (t, n, 0)),
            pl.BlockSpec((BP, 1), lambda t, n: (n, 0)),
            pl.BlockSpec((D, D), lambda t, n: (0, 0)),
            pl.BlockSpec((1, D), lambda t, n: (0, 0)),
        ],
        out_specs=pl.BlockSpec((1, D), lambda t, n: (t, 0)),
        out_shape=jax.ShapeDtypeStruct((T, D), jnp.float32),
    )(agg2, nd_col, W2, b2)


def _lstm_body(x_ref, wih_ref, whh_ref, b_ref, out_ref):
    wih = wih_ref[...]
    whh = whh_ref[...]
    b = b_ref[...]
    h = jnp.zeros((BN, H), jnp.float32)
    c = jnp.zeros((BN, H), jnp.float32)
    acc = jnp.zeros((BN, H), jnp.float32)
    for t in range(T):
        xt = x_ref[:, t, :]
        g = (jnp.dot(xt, wih, preferred_element_type=jnp.float32)
             + jnp.dot(h, whh, preferred_element_type=jnp.float32) + b)
        i = jax.nn.sigmoid(g[:, 0:H])
        f = jax.nn.sigmoid(g[:, H:2 * H])
        gg = jnp.tanh(g[:, 2 * H:3 * H])
        o = jax.nn.sigmoid(g[:, 3 * H:4 * H])
        c = f * c + i * gg
        h = o * jnp.tanh(c)
        acc = acc + h
    out_ref[...] = acc * (1.0 / T)


def _tc_lstm(h, W_ihT, W_hhT, bsum):
    grid = (N // BN,)
    return pl.pallas_call(
        _lstm_body,
        grid=grid,
        in_specs=[
            pl.BlockSpec((BN, T, D), lambda n: (n, 0, 0)),
            pl.BlockSpec((D, 4 * H), lambda n: (0, 0)),
            pl.BlockSpec((H, 4 * H), lambda n: (0, 0)),
            pl.BlockSpec((1, 4 * H), lambda n: (0, 0)),
        ],
        out_specs=pl.BlockSpec((BN, H), lambda n: (n, 0)),
        out_shape=jax.ShapeDtypeStruct((N, H), jnp.float32),
    )(h, W_ihT, W_hhT, bsum)


def kernel(h, edge_index, W1, b1, W2, b2, W_ih, W_hh, b_ih, b_hh):
    src = edge_index[0]
    dst = edge_index[1]
    src_deg = src.reshape(NC, NS, NCHUNK_D, CH)
    dst_deg = dst.reshape(NC, NS, NCHUNK_D, CH)
    src_r = src.reshape(NS, NCHUNK, CH)
    dst_r = dst.reshape(NS, NCHUNK, CH)

    deg = _sc_degrees(src_deg, dst_deg)
    norms = _tc_norms(deg)
    ns_col = norms[0].reshape(NPAD, 1)
    nd_col = norms[1].reshape(NPAD, 1)

    xs = _tc_prep(h, ns_col)
    agg1 = _sc_segsum(xs.reshape(NC * T * NPAD, HD), src_r, dst_r)
    ys = _tc_layer1(agg1, nd_col, ns_col, W1, b1)
    agg2 = _sc_segsum(ys.reshape(NC * T * NPAD, HD), src_r, dst_r)
    hs_out = _tc_layer2(agg2, nd_col, W2, b2).reshape(1, T, D)

    ht = _tc_lstm(h, W_ih.T, W_hh.T, (b_ih + b_hh).reshape(1, 4 * H))
    return (hs_out, ht)


# speedup vs baseline: 4.7362x; 4.7362x over previous
import functools

import jax
import jax.numpy as jnp
from jax import lax
from jax.experimental import pallas as pl
from jax.experimental.pallas import tpu as pltpu
from jax.experimental.pallas import tpu_sc as plsc

N = 10000
E = 320000
T = 12
D = 128
H = 128

NPAD = 10240
NC = 2
NS = 16
HD = D // NC

CH = 80
EPT = E // NS
NCHUNK = EPT // CH
EPW = E // (NC * NS)
NCHUNK_D = EPW // CH

ROWS_PT = NPAD // NS

_SC_MESH = plsc.VectorSubcoreMesh(core_axis_name="c", subcore_axis_name="s")


def _deg_body(src_hbm, dst_hbm, out_hbm, sbuf0, sbuf1, dbuf0, dbuf1,
              ones_v, zb, acc_s, acc_d, semi0, semi1, sems):
    c = lax.axis_index("c")
    s = lax.axis_index("s")

    one16 = jnp.ones((16,), jnp.float32)
    zero16 = jnp.zeros((16,), jnp.float32)
    for k in range(CH // 16):
        ones_v[pl.ds(k * 16, 16)] = one16

    def _zb(i, _):
        zb[pl.ds(i * 16, 16)] = zero16
        return 0
    lax.fori_loop(0, ROWS_PT // 16, _zb, 0)

    pltpu.sync_copy(zb, acc_s.at[pl.ds(s * ROWS_PT, ROWS_PT)])
    pltpu.sync_copy(zb, acc_d.at[pl.ds(s * ROWS_PT, ROWS_PT)])
    plsc.subcore_barrier()

    def _load(j, sb, db, semi):
        pltpu.async_copy(src_hbm.at[c, s, j], sb, semi)
        return pltpu.async_copy(dst_hbm.at[c, s, j], db, semi)

    def _wait(sb, db, semi):
        pltpu.make_async_copy(src_hbm.at[c, s, 0], sb, semi).wait()
        pltpu.make_async_copy(dst_hbm.at[c, s, 0], db, semi).wait()

    def _scat(sb, db):
        d0 = pltpu.async_copy(ones_v, acc_s.at[sb], sems, add=True)
        d1 = pltpu.async_copy(ones_v, acc_d.at[db], sems, add=True)
        d0.wait()
        d1.wait()

    _load(0, sbuf0, dbuf0, semi0)

    def _pair(m, _):
        _load(2 * m + 1, sbuf1, dbuf1, semi1)
        _wait(sbuf0, dbuf0, semi0)
        _scat(sbuf0, dbuf0)
        _load(2 * m + 2, sbuf0, dbuf0, semi0)
        _wait(sbuf1, dbuf1, semi1)
        _scat(sbuf1, dbuf1)
        return 0
    lax.fori_loop(0, (NCHUNK_D - 1) // 2, _pair, 0)
    _wait(sbuf0, dbuf0, semi0)
    _scat(sbuf0, dbuf0)

    plsc.subcore_barrier()
    pltpu.sync_copy(acc_s.at[pl.ds(s * ROWS_PT, ROWS_PT)],
                    out_hbm.at[c, 0, pl.ds(s * ROWS_PT, ROWS_PT)])
    pltpu.sync_copy(acc_d.at[pl.ds(s * ROWS_PT, ROWS_PT)],
                    out_hbm.at[c, 1, pl.ds(s * ROWS_PT, ROWS_PT)])


def _sc_degrees(src_r, dst_r):
    return pl.kernel(
        _deg_body,
        out_type=jax.ShapeDtypeStruct((NC, 2, NPAD), jnp.float32),
        mesh=_SC_MESH,
        scratch_types=[
            pltpu.VMEM((CH,), jnp.int32),
            pltpu.VMEM((CH,), jnp.int32),
            pltpu.VMEM((CH,), jnp.int32),
            pltpu.VMEM((CH,), jnp.int32),
            pltpu.VMEM((CH,), jnp.float32),
            pltpu.VMEM((ROWS_PT,), jnp.float32),
            pltpu.VMEM_SHARED((NPAD,), jnp.float32),
            pltpu.VMEM_SHARED((NPAD,), jnp.float32),
            pltpu.SemaphoreType.DMA,
            pltpu.SemaphoreType.DMA,
            pltpu.SemaphoreType.DMA,
        ],
    )(src_r, dst_r)


NPAIR = (NCHUNK_D - 1) // 2
ZROWS = 64


def _segsum_body(xflat, src_hbm, dst_hbm, out_hbm, sbuf0, sbuf1, dbuf0,
                 dbuf1, stage0, stage1, rows0, rows1, zb, acc,
                 semi0, semi1, semg0, semg1):
    c = lax.axis_index("c")
    s = lax.axis_index("s")

    zero16 = jnp.zeros((16,), jnp.float32)

    def _zb(i, _):
        r = i // (D // 16)
        k = i % (D // 16)
        zb[r, pl.ds(k * 16, 16)] = zero16
        return 0
    lax.fori_loop(0, ZROWS * (D // 16), _zb, 0)

    def _load(j, sb, db, semi):
        pltpu.async_copy(src_hbm.at[c, s, j], sb, semi)
        pltpu.async_copy(dst_hbm.at[c, s, j], db, semi)

    def _wait_idx(sb, db, semi):
        pltpu.make_async_copy(src_hbm.at[c, s, 0], sb, semi).wait()
        pltpu.make_async_copy(dst_hbm.at[c, s, 0], db, semi).wait()

    def _stage_fire(base, sb, stage, rows, semg):
        for k in range(CH // 16):
            v = sb[pl.ds(k * 16, 16)]
            stage[pl.ds(k * 16, 16)] = v + base
        pltpu.async_copy(xflat.at[stage], rows, semg)

    def _drain(db, stage, rows, semg):
        pltpu.make_async_copy(xflat.at[stage], rows, semg).wait()
        pltpu.sync_copy(rows, acc.at[db], add=True)

    def _per_t(t, _):
        base = t * NPAD
        for k in range(ROWS_PT // ZROWS):
            pltpu.sync_copy(zb, acc.at[pl.ds(s * ROWS_PT + k * ZROWS, ZROWS), :])
        plsc.subcore_barrier()

        _load(0, sbuf0, dbuf0, semi0)
        _wait_idx(sbuf0, dbuf0, semi0)
        _stage_fire(base, sbuf0, stage0, rows0, semg0)
        _load(1, sbuf1, dbuf1, semi1)

        def _pair(m, _):
            _wait_idx(sbuf1, dbuf1, semi1)
            _stage_fire(base, sbuf1, stage1, rows1, semg1)
            _drain(dbuf0, stage0, rows0, semg0)
            _load(2 * m + 2, sbuf0, dbuf0, semi0)
            _wait_idx(sbuf0, dbuf0, semi0)
            _stage_fire(base, sbuf0, stage0, rows0, semg0)
            _drain(dbuf1, stage1, rows1, semg1)

            @pl.when(m < NPAIR - 1)
            def _():
                _load(2 * m + 3, sbuf1, dbuf1, semi1)
            return 0

        lax.fori_loop(0, NPAIR, _pair, 0)
        _drain(dbuf0, stage0, rows0, semg0)
        plsc.subcore_barrier()
        pltpu.sync_copy(
            acc.at[pl.ds(s * ROWS_PT, ROWS_PT), :],
            out_hbm.at[c, t, pl.ds(s * ROWS_PT, ROWS_PT), :])
        return 0

    lax.fori_loop(0, T, _per_t, 0)


def _sc_segsum(xflat, src_r, dst_r):
    return pl.kernel(
        _segsum_body,
        out_type=jax.ShapeDtypeStruct((NC, T, NPAD, D), jnp.float32),
        mesh=_SC_MESH,
        scratch_types=[
            pltpu.VMEM((CH,), jnp.int32),
            pltpu.VMEM((CH,), jnp.int32),
            pltpu.VMEM((CH,), jnp.int32),
            pltpu.VMEM((CH,), jnp.int32),
            pltpu.VMEM((CH,), jnp.int32),
            pltpu.VMEM((CH,), jnp.int32),
            pltpu.VMEM((CH, D), jnp.float32),
            pltpu.VMEM((CH, D), jnp.float32),
            pltpu.VMEM((ZROWS, D), jnp.float32),
            pltpu.VMEM_SHARED((NPAD, D), jnp.float32),
            pltpu.SemaphoreType.DMA,
            pltpu.SemaphoreType.DMA,
            pltpu.SemaphoreType.DMA,
            pltpu.SemaphoreType.DMA,
        ],
    )(xflat, src_r, dst_r)


def _norms_body(deg_ref, out_ref):
    d = deg_ref[0] + deg_ref[1]
    out_ref[...] = lax.rsqrt(jnp.maximum(d, 1.0))


def _tc_norms(deg):
    return pl.pallas_call(
        _norms_body,
        out_shape=jax.ShapeDtypeStruct((2, NPAD), jnp.float32),
    )(deg)


BN = 2000
BP = 2048


def _prep_body(h_ref, ns_ref, out_ref):
    ns = ns_ref[...]
    for t in range(T):
        out_ref[t] = h_ref[:, t, :] * ns


BNP = 1000


def _tc_prep(h, norm_src_col):
    grid = (N // BNP,)
    return pl.pallas_call(
        _prep_body,
        grid=grid,
        in_specs=[
            pl.BlockSpec((BNP, T, D), lambda n: (n, 0, 0)),
            pl.BlockSpec((BNP, 1), lambda n: (n, 0)),
        ],
        out_specs=pl.BlockSpec((T, BNP, D), lambda n: (0, n, 0)),
        out_shape=jax.ShapeDtypeStruct((T, NPAD, D), jnp.float32),
    )(h, norm_src_col)


def _gelu(x):
    return 0.5 * x * (1.0 + lax.erf(x * 0.7071067811865476))


def _layer1_body(agg_ref, nd_ref, ns_ref, w_ref, b_ref, out_ref):
    agg = agg_ref[0, 0] + agg_ref[1, 0]
    z = agg * nd_ref[...]
    y = jnp.dot(z, w_ref[...], preferred_element_type=jnp.float32)
    out_ref[0] = _gelu(y + b_ref[...]) * ns_ref[...]


def _tc_layer1(agg1, nd_col, ns_col, W1, b1):
    grid = (T, NPAD // BP)
    return pl.pallas_call(
        _layer1_body,
        grid=grid,
        in_specs=[
            pl.BlockSpec((NC, 1, BP, D), lambda t, n: (0, t, n, 0)),
            pl.BlockSpec((BP, 1), lambda t, n: (n, 0)),
            pl.BlockSpec((BP, 1), lambda t, n: (n, 0)),
            pl.BlockSpec((D, D), lambda t, n: (0, 0)),
            pl.BlockSpec((1, D), lambda t, n: (0, 0)),
        ],
        out_specs=pl.BlockSpec((1, BP, D), lambda t, n: (t, n, 0)),
        out_shape=jax.ShapeDtypeStruct((T, NPAD, D), jnp.float32),
    )(agg1, nd_col, ns_col, W1, b1)


def _layer2_body(agg_ref, nd_ref, w_ref, b_ref, out_ref):
    t = pl.program_id(0)
    nb = pl.program_id(1)
    agg = agg_ref[0, 0] + agg_ref[1, 0]
    z = agg * nd_ref[...]
    y = jnp.dot(z, w_ref[...], preferred_element_type=jnp.float32)
    y = _gelu(y + b_ref[...])
    row = nb * BP + lax.broadcasted_iota(jnp.int32, (BP, 1), 0)
    y = jnp.where(row < N, y, 0.0)
    part = jnp.sum(y, axis=0, keepdims=True)

    @pl.when((t == 0) & (nb == 0))
    def _():
        out_ref[...] = jnp.zeros_like(out_ref)

    out_ref[pl.ds(t, 1), :] += part * (1.0 / N)


def _tc_layer2(agg2, nd_col, W2, b2):
    grid = (T, NPAD // BP)
    return pl.pallas_call(
        _layer2_body,
        grid=grid,
        in_specs=[
            pl.BlockSpec((NC, 1, BP, D), lambda t, n: (0, t, n, 0)),
            pl.BlockSpec((BP, 1), lambda t, n: (n, 0)),
            pl.BlockSpec((D, D), lambda t, n: (0, 0)),
            pl.BlockSpec((1, D), lambda t, n: (0, 0)),
        ],
        out_specs=pl.BlockSpec((T, D), lambda t, n: (0, 0)),
        out_shape=jax.ShapeDtypeStruct((T, D), jnp.float32),
    )(agg2, nd_col, W2, b2)


def _lstm_body(x_ref, wih_ref, whh_ref, b_ref, out_ref):
    wih = wih_ref[...]
    whh = whh_ref[...]
    b = b_ref[...]
    h = jnp.zeros((BN, H), jnp.float32)
    c = jnp.zeros((BN, H), jnp.float32)
    acc = jnp.zeros((BN, H), jnp.float32)
    for t in range(T):
        xt = x_ref[:, t, :]
        g = (jnp.dot(xt, wih, preferred_element_type=jnp.float32)
             + jnp.dot(h, whh, preferred_element_type=jnp.float32) + b)
        i = jax.nn.sigmoid(g[:, 0:H])
        f = jax.nn.sigmoid(g[:, H:2 * H])
        gg = jnp.tanh(g[:, 2 * H:3 * H])
        o = jax.nn.sigmoid(g[:, 3 * H:4 * H])
        c = f * c + i * gg
        h = o * jnp.tanh(c)
        acc = acc + h
    out_ref[...] = acc * (1.0 / T)


def _tc_lstm(h, W_ihT, W_hhT, bsum):
    grid = (N // BN,)
    return pl.pallas_call(
        _lstm_body,
        grid=grid,
        in_specs=[
            pl.BlockSpec((BN, T, D), lambda n: (n, 0, 0)),
            pl.BlockSpec((D, 4 * H), lambda n: (0, 0)),
            pl.BlockSpec((H, 4 * H), lambda n: (0, 0)),
            pl.BlockSpec((1, 4 * H), lambda n: (0, 0)),
        ],
        out_specs=pl.BlockSpec((BN, H), lambda n: (n, 0)),
        out_shape=jax.ShapeDtypeStruct((N, H), jnp.float32),
    )(h, W_ihT, W_hhT, bsum)


def kernel(h, edge_index, W1, b1, W2, b2, W_ih, W_hh, b_ih, b_hh):
    src = edge_index[0]
    dst = edge_index[1]
    src_deg = src.reshape(NC, NS, NCHUNK_D, CH)
    dst_deg = dst.reshape(NC, NS, NCHUNK_D, CH)

    deg = _sc_degrees(src_deg, dst_deg)
    norms = _tc_norms(deg)
    ns_col = norms[0].reshape(NPAD, 1)
    nd_col = norms[1].reshape(NPAD, 1)

    xs = _tc_prep(h, ns_col)
    agg1 = _sc_segsum(xs.reshape(T * NPAD, D), src_deg, dst_deg)
    ys = _tc_layer1(agg1, nd_col, ns_col, W1, b1.reshape(1, D))
    agg2 = _sc_segsum(ys.reshape(T * NPAD, D), src_deg, dst_deg)
    hs_out = _tc_layer2(agg2, nd_col, W2, b2.reshape(1, D)).reshape(1, T, D)

    ht = _tc_lstm(h, W_ih.T, W_hh.T, (b_ih + b_hh).reshape(1, 4 * H))
    return (hs_out, ht)


# speedup vs baseline: 5.4971x; 1.1607x over previous
import functools

import jax
import jax.numpy as jnp
from jax import lax
from jax.experimental import pallas as pl
from jax.experimental.pallas import tpu as pltpu
from jax.experimental.pallas import tpu_sc as plsc

N = 10000
E = 320000
T = 12
D = 128
H = 128

NPAD = 10240
NC = 2
NS = 16
HD = D // NC

CH = 80
EPT = E // NS
NCHUNK = EPT // CH
EPW = E // (NC * NS)
NCHUNK_D = EPW // CH

ROWS_PT = NPAD // NS

_SC_MESH = plsc.VectorSubcoreMesh(core_axis_name="c", subcore_axis_name="s")


def _deg_body(src_hbm, dst_hbm, out_hbm, sbuf0, sbuf1, dbuf0, dbuf1,
              ones_v, zb, acc_s, acc_d, semi0, semi1, sems):
    c = lax.axis_index("c")
    s = lax.axis_index("s")

    one16 = jnp.ones((16,), jnp.float32)
    zero16 = jnp.zeros((16,), jnp.float32)
    for k in range(CH // 16):
        ones_v[pl.ds(k * 16, 16)] = one16

    def _zb(i, _):
        zb[pl.ds(i * 16, 16)] = zero16
        return 0
    lax.fori_loop(0, ROWS_PT // 16, _zb, 0)

    pltpu.sync_copy(zb, acc_s.at[pl.ds(s * ROWS_PT, ROWS_PT)])
    pltpu.sync_copy(zb, acc_d.at[pl.ds(s * ROWS_PT, ROWS_PT)])
    plsc.subcore_barrier()

    def _load(j, sb, db, semi):
        pltpu.async_copy(src_hbm.at[c, s, j], sb, semi)
        return pltpu.async_copy(dst_hbm.at[c, s, j], db, semi)

    def _wait(sb, db, semi):
        pltpu.make_async_copy(src_hbm.at[c, s, 0], sb, semi).wait()
        pltpu.make_async_copy(dst_hbm.at[c, s, 0], db, semi).wait()

    def _scat(sb, db):
        d0 = pltpu.async_copy(ones_v, acc_s.at[sb], sems, add=True)
        d1 = pltpu.async_copy(ones_v, acc_d.at[db], sems, add=True)
        d0.wait()
        d1.wait()

    _load(0, sbuf0, dbuf0, semi0)

    def _pair(m, _):
        _load(2 * m + 1, sbuf1, dbuf1, semi1)
        _wait(sbuf0, dbuf0, semi0)
        _scat(sbuf0, dbuf0)
        _load(2 * m + 2, sbuf0, dbuf0, semi0)
        _wait(sbuf1, dbuf1, semi1)
        _scat(sbuf1, dbuf1)
        return 0
    lax.fori_loop(0, (NCHUNK_D - 1) // 2, _pair, 0)
    _wait(sbuf0, dbuf0, semi0)
    _scat(sbuf0, dbuf0)

    plsc.subcore_barrier()
    pltpu.sync_copy(acc_s.at[pl.ds(s * ROWS_PT, ROWS_PT)],
                    out_hbm.at[c, 0, pl.ds(s * ROWS_PT, ROWS_PT)])
    pltpu.sync_copy(acc_d.at[pl.ds(s * ROWS_PT, ROWS_PT)],
                    out_hbm.at[c, 1, pl.ds(s * ROWS_PT, ROWS_PT)])


def _sc_degrees(src_r, dst_r):
    return pl.kernel(
        _deg_body,
        out_type=jax.ShapeDtypeStruct((NC, 2, NPAD), jnp.float32),
        mesh=_SC_MESH,
        scratch_types=[
            pltpu.VMEM((CH,), jnp.int32),
            pltpu.VMEM((CH,), jnp.int32),
            pltpu.VMEM((CH,), jnp.int32),
            pltpu.VMEM((CH,), jnp.int32),
            pltpu.VMEM((CH,), jnp.float32),
            pltpu.VMEM((ROWS_PT,), jnp.float32),
            pltpu.VMEM_SHARED((NPAD,), jnp.float32),
            pltpu.VMEM_SHARED((NPAD,), jnp.float32),
            pltpu.SemaphoreType.DMA,
            pltpu.SemaphoreType.DMA,
            pltpu.SemaphoreType.DMA,
        ],
    )(src_r, dst_r)


NPAIR = (NCHUNK_D - 1) // 2
ZROWS = 64


def _segsum_body(xflat, src_hbm, dst_hbm, out_hbm, sbuf0, sbuf1, dbuf0,
                 dbuf1, dst0, dst1, stage0, stage1, rows0, rows1, zb, acc,
                 semi0, semi1, semg0, semg1, sems0, sems1):
    c = lax.axis_index("c")
    s = lax.axis_index("s")

    zero16 = jnp.zeros((16,), jnp.float32)

    def _zb(i, _):
        r = i // (D // 16)
        k = i % (D // 16)
        zb[r, pl.ds(k * 16, 16)] = zero16
        return 0
    lax.fori_loop(0, ZROWS * (D // 16), _zb, 0)

    def _load(j, sb, db, semi):
        pltpu.async_copy(src_hbm.at[c, s, j], sb, semi)
        pltpu.async_copy(dst_hbm.at[c, s, j], db, semi)

    def _wait_idx(sb, db, semi):
        pltpu.make_async_copy(src_hbm.at[c, s, 0], sb, semi).wait()
        pltpu.make_async_copy(dst_hbm.at[c, s, 0], db, semi).wait()

    def _stage_fire(base, sb, stage, rows, semg):
        for k in range(CH // 16):
            v = sb[pl.ds(k * 16, 16)]
            stage[pl.ds(k * 16, 16)] = v + base
        pltpu.async_copy(xflat.at[stage], rows, semg)

    def _fire_scatter(db, dstg, rows, sems):
        for k in range(CH // 16):
            dstg[pl.ds(k * 16, 16)] = db[pl.ds(k * 16, 16)]
        pltpu.async_copy(rows, acc.at[dstg], sems, add=True)

    def _wait_scatter(dstg, rows, sems):
        pltpu.make_async_copy(rows, acc.at[dstg], sems).wait()

    def _wait_gather(stage, rows, semg):
        pltpu.make_async_copy(xflat.at[stage], rows, semg).wait()

    def _per_t(t, _):
        base = t * NPAD
        for k in range(ROWS_PT // ZROWS):
            pltpu.sync_copy(zb, acc.at[pl.ds(s * ROWS_PT + k * ZROWS, ZROWS), :])
        plsc.subcore_barrier()

        _load(0, sbuf0, dbuf0, semi0)
        _wait_idx(sbuf0, dbuf0, semi0)
        _stage_fire(base, sbuf0, stage0, rows0, semg0)
        _load(1, sbuf1, dbuf1, semi1)

        def _half(m, jA, bufA, bufB, first):
            (sbA, dbA, dsA, stA, rwA, siA, sgA, ssA) = bufA
            (sbB, dbB, dsB, stB, rwB, siB, sgB, ssB) = bufB
            _wait_idx(sbB, dbB, siB)
            if first:
                @pl.when(m > 0)
                def _():
                    _wait_scatter(dsB, rwB, ssB)
            else:
                _wait_scatter(dsB, rwB, ssB)
            _stage_fire(base, sbB, stB, rwB, sgB)
            _wait_gather(stA, rwA, sgA)
            _fire_scatter(dbA, dsA, rwA, ssA)
            if first:
                _load(jA + 2, sbA, dbA, siA)
            else:
                @pl.when(m < NPAIR - 1)
                def _():
                    _load(jA + 2, sbA, dbA, siA)

        b0 = (sbuf0, dbuf0, dst0, stage0, rows0, semi0, semg0, sems0)
        b1 = (sbuf1, dbuf1, dst1, stage1, rows1, semi1, semg1, sems1)

        def _pair(m, _):
            _half(m, 2 * m, b0, b1, True)
            _half(m, 2 * m + 1, b1, b0, False)
            return 0

        lax.fori_loop(0, NPAIR, _pair, 0)
        _wait_gather(stage0, rows0, semg0)
        _fire_scatter(dbuf0, dst0, rows0, sems0)
        _wait_scatter(dst1, rows1, sems1)
        _wait_scatter(dst0, rows0, sems0)
        plsc.subcore_barrier()
        pltpu.sync_copy(
            acc.at[pl.ds(s * ROWS_PT, ROWS_PT), :],
            out_hbm.at[c, t, pl.ds(s * ROWS_PT, ROWS_PT), :])
        return 0

    lax.fori_loop(0, T, _per_t, 0)


def _sc_segsum(xflat, src_r, dst_r):
    return pl.kernel(
        _segsum_body,
        out_type=jax.ShapeDtypeStruct((NC, T, NPAD, D), jnp.float32),
        mesh=_SC_MESH,
        scratch_types=[
            pltpu.VMEM((CH,), jnp.int32),
            pltpu.VMEM((CH,), jnp.int32),
            pltpu.VMEM((CH,), jnp.int32),
            pltpu.VMEM((CH,), jnp.int32),
            pltpu.VMEM((CH,), jnp.int32),
            pltpu.VMEM((CH,), jnp.int32),
            pltpu.VMEM((CH,), jnp.int32),
            pltpu.VMEM((CH,), jnp.int32),
            pltpu.VMEM((CH, D), jnp.float32),
            pltpu.VMEM((CH, D), jnp.float32),
            pltpu.VMEM((ZROWS, D), jnp.float32),
            pltpu.VMEM_SHARED((NPAD, D), jnp.float32),
            pltpu.SemaphoreType.DMA,
            pltpu.SemaphoreType.DMA,
            pltpu.SemaphoreType.DMA,
            pltpu.SemaphoreType.DMA,
            pltpu.SemaphoreType.DMA,
            pltpu.SemaphoreType.DMA,
        ],
    )(xflat, src_r, dst_r)


def _norms_body(deg_ref, out_ref):
    d = deg_ref[0] + deg_ref[1]
    out_ref[...] = lax.rsqrt(jnp.maximum(d, 1.0))


def _tc_norms(deg):
    return pl.pallas_call(
        _norms_body,
        out_shape=jax.ShapeDtypeStruct((2, NPAD), jnp.float32),
    )(deg)


BN = 2000
BP = 2048


def _prep_body(h_ref, ns_ref, out_ref):
    ns = ns_ref[...]
    for t in range(T):
        out_ref[t] = h_ref[:, t, :] * ns


BNP = 1000


def _tc_prep(h, norm_src_col):
    grid = (N // BNP,)
    return pl.pallas_call(
        _prep_body,
        grid=grid,
        in_specs=[
            pl.BlockSpec((BNP, T, D), lambda n: (n, 0, 0)),
            pl.BlockSpec((BNP, 1), lambda n: (n, 0)),
        ],
        out_specs=pl.BlockSpec((T, BNP, D), lambda n: (0, n, 0)),
        out_shape=jax.ShapeDtypeStruct((T, NPAD, D), jnp.float32),
    )(h, norm_src_col)


def _gelu(x):
    return 0.5 * x * (1.0 + lax.erf(x * 0.7071067811865476))


def _layer1_body(agg_ref, nd_ref, ns_ref, w_ref, b_ref, out_ref):
    agg = agg_ref[0, 0] + agg_ref[1, 0]
    z = agg * nd_ref[...]
    y = jnp.dot(z, w_ref[...], preferred_element_type=jnp.float32)
    out_ref[0] = _gelu(y + b_ref[...]) * ns_ref[...]


def _tc_layer1(agg1, nd_col, ns_col, W1, b1):
    grid = (T, NPAD // BP)
    return pl.pallas_call(
        _layer1_body,
        grid=grid,
        in_specs=[
            pl.BlockSpec((NC, 1, BP, D), lambda t, n: (0, t, n, 0)),
            pl.BlockSpec((BP, 1), lambda t, n: (n, 0)),
            pl.BlockSpec((BP, 1), lambda t, n: (n, 0)),
            pl.BlockSpec((D, D), lambda t, n: (0, 0)),
            pl.BlockSpec((1, D), lambda t, n: (0, 0)),
        ],
        out_specs=pl.BlockSpec((1, BP, D), lambda t, n: (t, n, 0)),
        out_shape=jax.ShapeDtypeStruct((T, NPAD, D), jnp.float32),
    )(agg1, nd_col, ns_col, W1, b1)


def _layer2_body(agg_ref, nd_ref, w_ref, b_ref, out_ref):
    t = pl.program_id(0)
    nb = pl.program_id(1)
    agg = agg_ref[0, 0] + agg_ref[1, 0]
    z = agg * nd_ref[...]
    y = jnp.dot(z, w_ref[...], preferred_element_type=jnp.float32)
    y = _gelu(y + b_ref[...])
    row = nb * BP + lax.broadcasted_iota(jnp.int32, (BP, 1), 0)
    y = jnp.where(row < N, y, 0.0)
    part = jnp.sum(y, axis=0, keepdims=True)

    @pl.when((t == 0) & (nb == 0))
    def _():
        out_ref[...] = jnp.zeros_like(out_ref)

    out_ref[pl.ds(t, 1), :] += part * (1.0 / N)


def _tc_layer2(agg2, nd_col, W2, b2):
    grid = (T, NPAD // BP)
    return pl.pallas_call(
        _layer2_body,
        grid=grid,
        in_specs=[
            pl.BlockSpec((NC, 1, BP, D), lambda t, n: (0, t, n, 0)),
            pl.BlockSpec((BP, 1), lambda t, n: (n, 0)),
            pl.BlockSpec((D, D), lambda t, n: (0, 0)),
            pl.BlockSpec((1, D), lambda t, n: (0, 0)),
        ],
        out_specs=pl.BlockSpec((T, D), lambda t, n: (0, 0)),
        out_shape=jax.ShapeDtypeStruct((T, D), jnp.float32),
    )(agg2, nd_col, W2, b2)


def _lstm_body(x_ref, wih_ref, whh_ref, b_ref, out_ref):
    wih = wih_ref[...]
    whh = whh_ref[...]
    b = b_ref[...]
    h = jnp.zeros((BN, H), jnp.float32)
    c = jnp.zeros((BN, H), jnp.float32)
    acc = jnp.zeros((BN, H), jnp.float32)
    for t in range(T):
        xt = x_ref[:, t, :]
        g = (jnp.dot(xt, wih, preferred_element_type=jnp.float32)
             + jnp.dot(h, whh, preferred_element_type=jnp.float32) + b)
        i = jax.nn.sigmoid(g[:, 0:H])
        f = jax.nn.sigmoid(g[:, H:2 * H])
        gg = jnp.tanh(g[:, 2 * H:3 * H])
        o = jax.nn.sigmoid(g[:, 3 * H:4 * H])
        c = f * c + i * gg
        h = o * jnp.tanh(c)
        acc = acc + h
    out_ref[...] = acc * (1.0 / T)


def _tc_lstm(h, W_ihT, W_hhT, bsum):
    grid = (N // BN,)
    return pl.pallas_call(
        _lstm_body,
        grid=grid,
        in_specs=[
            pl.BlockSpec((BN, T, D), lambda n: (n, 0, 0)),
            pl.BlockSpec((D, 4 * H), lambda n: (0, 0)),
            pl.BlockSpec((H, 4 * H), lambda n: (0, 0)),
            pl.BlockSpec((1, 4 * H), lambda n: (0, 0)),
        ],
        out_specs=pl.BlockSpec((BN, H), lambda n: (n, 0)),
        out_shape=jax.ShapeDtypeStruct((N, H), jnp.float32),
    )(h, W_ihT, W_hhT, bsum)


def kernel(h, edge_index, W1, b1, W2, b2, W_ih, W_hh, b_ih, b_hh):
    src = edge_index[0]
    dst = edge_index[1]
    src_deg = src.reshape(NC, NS, NCHUNK_D, CH)
    dst_deg = dst.reshape(NC, NS, NCHUNK_D, CH)

    deg = _sc_degrees(src_deg, dst_deg)
    norms = _tc_norms(deg)
    ns_col = norms[0].reshape(NPAD, 1)
    nd_col = norms[1].reshape(NPAD, 1)

    xs = _tc_prep(h, ns_col)
    agg1 = _sc_segsum(xs.reshape(T * NPAD, D), src_deg, dst_deg)
    ys = _tc_layer1(agg1, nd_col, ns_col, W1, b1.reshape(1, D))
    agg2 = _sc_segsum(ys.reshape(T * NPAD, D), src_deg, dst_deg)
    hs_out = _tc_layer2(agg2, nd_col, W2, b2.reshape(1, D)).reshape(1, T, D)

    ht = _tc_lstm(h, W_ih.T, W_hh.T, (b_ih + b_hh).reshape(1, 4 * H))
    return (hs_out, ht)
